# Initial kernel scaffold; baseline (speedup 1.0000x reference)
import jax
import jax.numpy as jnp
from jax import lax
from jax.experimental import pallas as pl
from jax.experimental.pallas import tpu as pltpu
from jax.experimental.pallas import tpu_sc as plsc

N = 10000
D = 128
E = 320000
NC = 2
NS = 16
NW = NC * NS
EB = 128
NB = -(-E // (NW * EB))
EPW = NB * EB
EP = EPW * NW
ROWS_PER_TILE = N // NS

_Z16 = jnp.zeros((16,), jnp.float32)


def _worker_id():
    c = lax.axis_index("c")
    s = lax.axis_index("s")
    return c, s, c * NS + s


def _deg_body(cols_hbm, ew_hbm, degp_hbm, col_v, ew_v, zb, deg_sh):
    c, s, wid = _worker_id()
    for i in range(8):
        zb[pl.ds(i * 16, 16)] = _Z16
    off = s * ROWS_PER_TILE
    for j in range(ROWS_PER_TILE // EB):
        pltpu.sync_copy(zb, deg_sh.at[pl.ds(off + j * EB, EB)])
    rem = ROWS_PER_TILE % EB
    if rem:
        pltpu.sync_copy(zb.at[pl.ds(0, rem)],
                        deg_sh.at[pl.ds(off + (ROWS_PER_TILE // EB) * EB, rem)])
    plsc.subcore_barrier()

    pltpu.sync_copy(cols_hbm.at[wid], col_v)
    pltpu.sync_copy(ew_hbm.at[wid], ew_v)

    def blk(j, carry):
        pltpu.sync_copy(ew_v.at[j], deg_sh.at[col_v.at[j]], add=True)
        return carry

    lax.fori_loop(0, NB, blk, 0)
    plsc.subcore_barrier()

    @pl.when(s == 0)
    def _():
        pltpu.sync_copy(deg_sh, degp_hbm.at[c])


def _sc_deg(cols3, ew3):
    mesh = plsc.VectorSubcoreMesh(core_axis_name="c", subcore_axis_name="s")
    f = pl.kernel(
        _deg_body,
        out_type=jax.ShapeDtypeStruct((NC, N), jnp.float32),
        mesh=mesh,
        scratch_types=[
            pltpu.VMEM((NB, EB), jnp.int32),
            pltpu.VMEM((NB, EB), jnp.float32),
            pltpu.VMEM((EB,), jnp.float32),
            pltpu.VMEM_SHARED((N,), jnp.float32),
        ],
    )
    return f(cols3, ew3)


def _spmm_body(rows_hbm, cols_hbm, ew_hbm, y_hbm, outp_hbm,
               row_v, col_v, ew_v, gbuf, sem, out_sh):
    c, s, wid = _worker_id()
    off = s * ROWS_PER_TILE

    def zrow(e, carry):
        for k in range(8):
            gbuf[e, pl.ds(k * 16, 16)] = _Z16
        return carry

    lax.fori_loop(0, EB, zrow, 0)
    for j in range(ROWS_PER_TILE // EB):
        pltpu.sync_copy(gbuf, out_sh.at[pl.ds(off + j * EB, EB)])
    rem = ROWS_PER_TILE % EB
    if rem:
        pltpu.sync_copy(gbuf.at[pl.ds(0, rem)],
                        out_sh.at[pl.ds(off + (ROWS_PER_TILE // EB) * EB, rem)])
    plsc.subcore_barrier()

    pltpu.sync_copy(rows_hbm.at[wid], row_v)
    pltpu.sync_copy(cols_hbm.at[wid], col_v)
    pltpu.sync_copy(ew_hbm.at[wid], ew_v)

    def blk(j, carry):
        pltpu.async_copy(y_hbm.at[row_v.at[j]], gbuf, sem).wait()

        def scale(e, cc):
            w = ew_v[j, e]
            for k in range(8):
                sl = pl.ds(k * 16, 16)
                gbuf[e, sl] = gbuf[e, sl] * w
            return cc

        lax.fori_loop(0, EB, scale, 0)
        pltpu.sync_copy(gbuf, out_sh.at[col_v.at[j]], add=True)
        return carry

    lax.fori_loop(0, NB, blk, 0)
    plsc.subcore_barrier()

    pltpu.sync_copy(out_sh.at[pl.ds(off, ROWS_PER_TILE)],
                    outp_hbm.at[c, pl.ds(off, ROWS_PER_TILE)])


def _sc_spmm(rows3, cols3, ew3, y):
    mesh = plsc.VectorSubcoreMesh(core_axis_name="c", subcore_axis_name="s")
    f = pl.kernel(
        _spmm_body,
        out_type=jax.ShapeDtypeStruct((NC, N, D), jnp.float32),
        mesh=mesh,
        scratch_types=[
            pltpu.VMEM((NB, EB), jnp.int32),
            pltpu.VMEM((NB, EB), jnp.int32),
            pltpu.VMEM((NB, EB), jnp.float32),
            pltpu.VMEM((EB, D), jnp.float32),
            pltpu.SemaphoreType.DMA,
            pltpu.VMEM_SHARED((N, D), jnp.float32),
        ],
    )
    return f(rows3, cols3, ew3, y)


BN = 1000


def _prep_body(degp_ref, x_ref, w_ref, dinv_ref, y_ref):
    dp = degp_ref[...]
    deg = dp[0] + dp[1] + 1.0
    dinv = lax.rsqrt(jnp.maximum(deg, 1e-12))
    xw = jnp.dot(x_ref[...], w_ref[...], preferred_element_type=jnp.float32)
    dinv_ref[...] = dinv
    y_ref[...] = dinv * xw


def _tc_prep(degp, x, W):
    degp3 = degp.reshape(NC, N, 1)
    return pl.pallas_call(
        _prep_body,
        grid=(N // BN,),
        in_specs=[
            pl.BlockSpec((NC, BN, 1), lambda i: (0, i, 0)),
            pl.BlockSpec((BN, D), lambda i: (i, 0)),
            pl.BlockSpec((D, D), lambda i: (0, 0)),
        ],
        out_specs=[
            pl.BlockSpec((BN, 1), lambda i: (i, 0)),
            pl.BlockSpec((BN, D), lambda i: (i, 0)),
        ],
        out_shape=[
            jax.ShapeDtypeStruct((N, 1), jnp.float32),
            jax.ShapeDtypeStruct((N, D), jnp.float32),
        ],
    )(degp3, x, W)


def _mid_body(sp_ref, dinv_ref, y_ref, b_ref, w_ref, y2_ref):
    sp = sp_ref[0] + sp_ref[1] + y_ref[...]
    dinv = dinv_ref[...]
    h = jnp.maximum(dinv * sp + b_ref[...], 0.0)
    xw2 = jnp.dot(h, w_ref[...], preferred_element_type=jnp.float32)
    y2_ref[...] = dinv * xw2


def _tc_mid(sp, dinv, y, b, W):
    return pl.pallas_call(
        _mid_body,
        grid=(N // BN,),
        in_specs=[
            pl.BlockSpec((NC, BN, D), lambda i: (0, i, 0)),
            pl.BlockSpec((BN, 1), lambda i: (i, 0)),
            pl.BlockSpec((BN, D), lambda i: (i, 0)),
            pl.BlockSpec((1, D), lambda i: (0, 0)),
            pl.BlockSpec((D, D), lambda i: (0, 0)),
        ],
        out_specs=pl.BlockSpec((BN, D), lambda i: (i, 0)),
        out_shape=jax.ShapeDtypeStruct((N, D), jnp.float32),
    )(sp, dinv, y, b.reshape(1, D), W)


def _final_body(sp_ref, dinv_ref, y_ref, b_ref, out_ref):
    sp = sp_ref[0] + sp_ref[1] + y_ref[...]
    out_ref[...] = dinv_ref[...] * sp + b_ref[...]


def _tc_final(sp, dinv, y, b):
    return pl.pallas_call(
        _final_body,
        grid=(N // BN,),
        in_specs=[
            pl.BlockSpec((NC, BN, D), lambda i: (0, i, 0)),
            pl.BlockSpec((BN, 1), lambda i: (i, 0)),
            pl.BlockSpec((BN, D), lambda i: (i, 0)),
            pl.BlockSpec((1, D), lambda i: (0, 0)),
        ],
        out_specs=pl.BlockSpec((BN, D), lambda i: (i, 0)),
        out_shape=jax.ShapeDtypeStruct((N, D), jnp.float32),
    )(sp, dinv, y, b.reshape(1, D))


@jax.jit
def kernel(x, edge_index, edge_attr, W1, b1, W2, b2):
    pad = EP - E
    rows3 = jnp.pad(edge_index[0], (0, pad)).reshape(NW, NB, EB)
    cols3 = jnp.pad(edge_index[1], (0, pad)).reshape(NW, NB, EB)
    ew3 = jnp.pad(edge_attr, (0, pad)).reshape(NW, NB, EB)

    degp = _sc_deg(cols3, ew3)
    dinv, y1 = _tc_prep(degp, x, W1)
    s1 = _sc_spmm(rows3, cols3, ew3, y1)
    y2 = _tc_mid(s1, dinv, y1, b1, W2)
    s2 = _sc_spmm(rows3, cols3, ew3, y2)
    return _tc_final(s2, dinv, y2, b2)


# speedup vs baseline: 11.9314x; 11.9314x over previous
import jax
import jax.numpy as jnp
from jax import lax
from jax.experimental import pallas as pl
from jax.experimental.pallas import tpu as pltpu
from jax.experimental.pallas import tpu_sc as plsc

N = 10000
D = 128
E = 320000
NC = 2
NS = 16
NW = NC * NS
EB = 128
NB = -(-E // (NW * EB))
EPW = NB * EB
EP = EPW * NW
ROWS_PER_TILE = N // NS

def _z16():
    return jnp.zeros((16,), jnp.float32)


def _worker_id():
    c = lax.axis_index("c")
    s = lax.axis_index("s")
    return c, s, c * NS + s


def _deg_body(cols_hbm, ew_hbm, degp_hbm, col_v, ew_v, zb, deg_sh):
    c, s, wid = _worker_id()
    for i in range(8):
        zb[pl.ds(i * 16, 16)] = _z16()

    @pl.when(s == 0)
    def _():
        def zc(jj, cc):
            pltpu.sync_copy(zb, deg_sh.at[pl.ds(jj * EB, EB)])
            return cc

        lax.fori_loop(0, N // EB, zc, 0)
        pltpu.sync_copy(zb.at[pl.ds(0, N % EB)],
                        deg_sh.at[pl.ds((N // EB) * EB, N % EB)])

    plsc.subcore_barrier()

    pltpu.sync_copy(cols_hbm.at[wid], col_v)
    pltpu.sync_copy(ew_hbm.at[wid], ew_v)

    def blk(j, carry):
        pltpu.sync_copy(ew_v.at[j], deg_sh.at[col_v.at[j]], add=True)
        return carry

    lax.fori_loop(0, NB, blk, 0)
    plsc.subcore_barrier()

    @pl.when(s == 0)
    def _():
        pltpu.sync_copy(deg_sh, degp_hbm.at[c])


def _sc_deg(cols3, ew3):
    mesh = plsc.VectorSubcoreMesh(core_axis_name="c", subcore_axis_name="s")
    f = pl.kernel(
        _deg_body,
        out_type=jax.ShapeDtypeStruct((NC, N), jnp.float32),
        mesh=mesh,
        scratch_types=[
            pltpu.VMEM((NB, EB), jnp.int32),
            pltpu.VMEM((NB, EB), jnp.float32),
            pltpu.VMEM((EB,), jnp.float32),
            pltpu.VMEM_SHARED((N,), jnp.float32),
        ],
    )
    return f(cols3, ew3)


def _spmm_body(rows_hbm, cols_hbm, ew_hbm, y_hbm, outp_hbm,
               row_v, col_v, ew_v, gbuf, sem, out_sh):
    c, s, wid = _worker_id()
    base = s * 624

    def zrow(e, carry):
        for k in range(8):
            gbuf[e, pl.ds(k * 16, 16)] = _z16()
        return carry

    lax.fori_loop(0, EB, zrow, 0)

    def _zero_rows(b0, cnt):
        for j in range(cnt // EB):
            pltpu.sync_copy(gbuf, out_sh.at[pl.ds(b0 + j * EB, EB)])
        rem = cnt % EB
        if rem:
            pltpu.sync_copy(gbuf.at[pl.ds(0, rem)],
                            out_sh.at[pl.ds(b0 + (cnt // EB) * EB, rem)])

    @pl.when(s < NS - 1)
    def _():
        _zero_rows(base, 624)

    @pl.when(s == NS - 1)
    def _():
        _zero_rows(base, 640)

    plsc.subcore_barrier()

    pltpu.sync_copy(rows_hbm.at[wid], row_v)
    pltpu.sync_copy(cols_hbm.at[wid], col_v)
    pltpu.sync_copy(ew_hbm.at[wid], ew_v)

    def blk(j, carry):
        pltpu.async_copy(y_hbm.at[row_v.at[j]], gbuf, sem).wait()

        def scale_grp(g, cc):
            wv = ew_v[j, pl.ds(g * 16, 16)]
            for t in range(16):
                w = wv[t]
                e = g * 16 + t
                for k in range(8):
                    sl = pl.ds(k * 16, 16)
                    gbuf[e, sl] = gbuf[e, sl] * w
            return cc

        lax.fori_loop(0, EB // 16, scale_grp, 0)
        pltpu.sync_copy(gbuf, out_sh.at[col_v.at[j]], add=True)
        return carry

    lax.fori_loop(0, NB, blk, 0)
    plsc.subcore_barrier()

    @pl.when(s < NS - 1)
    def _():
        pltpu.sync_copy(out_sh.at[pl.ds(base, 624)],
                        outp_hbm.at[c, pl.ds(base, 624)])

    @pl.when(s == NS - 1)
    def _():
        pltpu.sync_copy(out_sh.at[pl.ds(base, 640)],
                        outp_hbm.at[c, pl.ds(base, 640)])


def _sc_spmm(rows3, cols3, ew3, y):
    mesh = plsc.VectorSubcoreMesh(core_axis_name="c", subcore_axis_name="s")
    f = pl.kernel(
        _spmm_body,
        out_type=jax.ShapeDtypeStruct((NC, N, D), jnp.float32),
        mesh=mesh,
        scratch_types=[
            pltpu.VMEM((NB, EB), jnp.int32),
            pltpu.VMEM((NB, EB), jnp.int32),
            pltpu.VMEM((NB, EB), jnp.float32),
            pltpu.VMEM((EB, D), jnp.float32),
            pltpu.SemaphoreType.DMA,
            pltpu.VMEM_SHARED((N, D), jnp.float32),
        ],
    )
    return f(rows3, cols3, ew3, y)


BN = 1000


def _prep_body(degp_ref, x_ref, w_ref, dinv_ref, y_ref):
    dp = degp_ref[...]
    deg = dp[0] + dp[1] + 1.0
    dinv = lax.rsqrt(jnp.maximum(deg, 1e-12))
    xw = jnp.dot(x_ref[...], w_ref[...], preferred_element_type=jnp.float32)
    dinv_ref[...] = dinv
    y_ref[...] = dinv * xw


def _tc_prep(degp, x, W):
    degp3 = degp.reshape(NC, N, 1)
    return pl.pallas_call(
        _prep_body,
        grid=(N // BN,),
        in_specs=[
            pl.BlockSpec((NC, BN, 1), lambda i: (0, i, 0)),
            pl.BlockSpec((BN, D), lambda i: (i, 0)),
            pl.BlockSpec((D, D), lambda i: (0, 0)),
        ],
        out_specs=[
            pl.BlockSpec((BN, 1), lambda i: (i, 0)),
            pl.BlockSpec((BN, D), lambda i: (i, 0)),
        ],
        out_shape=[
            jax.ShapeDtypeStruct((N, 1), jnp.float32),
            jax.ShapeDtypeStruct((N, D), jnp.float32),
        ],
    )(degp3, x, W)


def _mid_body(sp_ref, dinv_ref, y_ref, b_ref, w_ref, y2_ref):
    sp = sp_ref[0] + sp_ref[1] + y_ref[...]
    dinv = dinv_ref[...]
    h = jnp.maximum(dinv * sp + b_ref[...], 0.0)
    xw2 = jnp.dot(h, w_ref[...], preferred_element_type=jnp.float32)
    y2_ref[...] = dinv * xw2


def _tc_mid(sp, dinv, y, b, W):
    return pl.pallas_call(
        _mid_body,
        grid=(N // BN,),
        in_specs=[
            pl.BlockSpec((NC, BN, D), lambda i: (0, i, 0)),
            pl.BlockSpec((BN, 1), lambda i: (i, 0)),
            pl.BlockSpec((BN, D), lambda i: (i, 0)),
            pl.BlockSpec((1, D), lambda i: (0, 0)),
            pl.BlockSpec((D, D), lambda i: (0, 0)),
        ],
        out_specs=pl.BlockSpec((BN, D), lambda i: (i, 0)),
        out_shape=jax.ShapeDtypeStruct((N, D), jnp.float32),
    )(sp, dinv, y, b.reshape(1, D), W)


def _final_body(sp_ref, dinv_ref, y_ref, b_ref, out_ref):
    sp = sp_ref[0] + sp_ref[1] + y_ref[...]
    out_ref[...] = dinv_ref[...] * sp + b_ref[...]


def _tc_final(sp, dinv, y, b):
    return pl.pallas_call(
        _final_body,
        grid=(N // BN,),
        in_specs=[
            pl.BlockSpec((NC, BN, D), lambda i: (0, i, 0)),
            pl.BlockSpec((BN, 1), lambda i: (i, 0)),
            pl.BlockSpec((BN, D), lambda i: (i, 0)),
            pl.BlockSpec((1, D), lambda i: (0, 0)),
        ],
        out_specs=pl.BlockSpec((BN, D), lambda i: (i, 0)),
        out_shape=jax.ShapeDtypeStruct((N, D), jnp.float32),
    )(sp, dinv, y, b.reshape(1, D))


@jax.jit
def kernel(x, edge_index, edge_attr, W1, b1, W2, b2):
    pad = EP - E
    rows3 = jnp.pad(edge_index[0], (0, pad)).reshape(NW, NB, EB)
    cols3 = jnp.pad(edge_index[1], (0, pad)).reshape(NW, NB, EB)
    ew3 = jnp.pad(edge_attr, (0, pad)).reshape(NW, NB, EB)

    degp = _sc_deg(cols3, ew3)
    dinv, y1 = _tc_prep(degp, x, W1)
    s1 = _sc_spmm(rows3, cols3, ew3, y1)
    y2 = _tc_mid(s1, dinv, y1, b1, W2)
    s2 = _sc_spmm(rows3, cols3, ew3, y2)
    return _tc_final(s2, dinv, y2, b2)
